# baseline (speedup 1.0000x reference)
import functools

import jax
import jax.numpy as jnp
from jax import lax
from jax.experimental import pallas as pl
from jax.experimental.pallas import tpu as pltpu
from jax.experimental.pallas import tpu_sc as plsc

NC = 2
NS = 16
NW = NC * NS
L = 16
D = 64
QROWS = 1 << 18
QMASK = QROWS - 1
HW = D // 2


def _rsqrt16(x):
    i = plsc.bitcast(x, jnp.int32)
    z = plsc.bitcast(jnp.int32(0x5F3759DF) - (i >> 1), jnp.float32)
    for _ in range(3):
        z = z * (1.5 - 0.5 * x * z * z)
    return z


def _build_sc_kernel(B):
    CH = B // NW
    NSUB = 2
    SUB = CH // NSUB
    NG = SUB // L

    mesh = plsc.VectorSubcoreMesh(
        core_axis_name="c", subcore_axis_name="s",
        num_cores=NC, num_subcores=NS)

    @functools.partial(
        pl.kernel,
        out_type=jax.ShapeDtypeStruct((NW, L), jnp.float32),
        mesh=mesh,
        scratch_types=[
            pltpu.VMEM((SUB,), jnp.int32),
            pltpu.VMEM((SUB,), jnp.int32),
            pltpu.VMEM((SUB,), jnp.int32),
            pltpu.VMEM((SUB,), jnp.int32),
            pltpu.VMEM((SUB,), jnp.int32),
            pltpu.VMEM((SUB,), jnp.int32),
            pltpu.VMEM((SUB,), jnp.int32),
            pltpu.VMEM((SUB, 2 * D), jnp.int32),
            pltpu.VMEM((SUB, 2 * D), jnp.int32),
            pltpu.VMEM((SUB, 2 * D), jnp.int32),
            pltpu.VMEM((CH,), jnp.float32),
            pltpu.VMEM((L,), jnp.float32),
            pltpu.SemaphoreType.DMA,
        ],
        compiler_params=pltpu.CompilerParams(
            needs_layout_passes=False, use_tc_tiling_on_sc=False),
    )
    def sc_kernel(ph, pr, pt, nh, nr, nt, node_pk, edge_pk, out,
                  raw_v, rh_v, rr_v, rt_v, ch_v, cr_v, ct_v,
                  h_v, r_v, t_v, pd_v, acc_v, sem):
        wid = lax.axis_index("s") * NC + lax.axis_index("c")

        def stage_idx(ix_hbm, base, row_v, cb_v):
            pltpu.sync_copy(ix_hbm.at[pl.ds(base, SUB)], raw_v)
            for k in range(SUB // L):
                v = raw_v[pl.ds(k * L, L)]
                row_v[pl.ds(k * L, L)] = v & QMASK
                cb_v[pl.ds(k * L, L)] = (v >> 18) << 5

        def gather_sub(hi, ri, ti, base):
            stage_idx(hi, base, rh_v, ch_v)
            stage_idx(ri, base, rr_v, cr_v)
            stage_idx(ti, base, rt_v, ct_v)
            c1 = pltpu.async_copy(node_pk.at[rh_v], h_v, sem)
            c2 = pltpu.async_copy(edge_pk.at[rr_v], r_v, sem)
            c3 = pltpu.async_copy(node_pk.at[rt_v], t_v, sem)
            c1.wait(); c2.wait(); c3.wait()

        def dist_group(g):
            row = g * L + lax.iota(jnp.int32, L)
            cbh = ch_v[pl.ds(g * L, L)]
            cbr = cr_v[pl.ds(g * L, L)]
            cbt = ct_v[pl.ds(g * L, L)]
            fmt = plsc.PackFormat.INTERLEAVED

            def jstep(w, ss):
                hw = plsc.load_gather(h_v, [row, cbh + w])
                rw = plsc.load_gather(r_v, [row, cbr + w])
                tw = plsc.load_gather(t_v, [row, cbt + w])
                ha, hb = plsc.unpack(plsc.bitcast(hw, jnp.bfloat16), format=fmt)
                ra, rb = plsc.unpack(plsc.bitcast(rw, jnp.bfloat16), format=fmt)
                ta, tb = plsc.unpack(plsc.bitcast(tw, jnp.bfloat16), format=fmt)
                d1 = ha + ra - ta
                d2 = hb + rb - tb
                return ss + d1 * d1 + d2 * d2

            ss = lax.fori_loop(0, HW, jstep, jnp.zeros((L,), jnp.float32))
            return ss * _rsqrt16(ss)

        base0 = wid * CH
        for s in range(NSUB):
            gather_sub(ph, pr, pt, base0 + s * SUB)

            def pos_body(g, carry, s=s):
                pd_v[pl.ds(s * SUB + g * L, L)] = dist_group(g)
                return carry

            lax.fori_loop(0, NG, pos_body, 0)

        acc = jnp.zeros((L,), jnp.float32)
        for s in range(NSUB):
            gather_sub(nh, nr, nt, base0 + s * SUB)

            def neg_body(g, a, s=s):
                nd = dist_group(g)
                pd = pd_v[pl.ds(s * SUB + g * L, L)]
                return a + jnp.maximum(pd - nd, 0.0)

            acc = lax.fori_loop(0, NG, neg_body, acc)

        acc_v[...] = acc
        pltpu.sync_copy(acc_v, out.at[wid])

    return sc_kernel


def _finish_body(inv_b, p_ref, o_ref):
    o_ref[...] = jnp.reshape(jnp.sum(p_ref[...]) * inv_b, (1, 1))


_TBLK = 8192


def _tr_body(x0_ref, x1_ref, x2_ref, x3_ref, o_ref):
    d = x0_ref.shape[0]
    eye = (lax.broadcasted_iota(jnp.int32, (d, d), 0)
           == lax.broadcasted_iota(jnp.int32, (d, d), 1)).astype(jnp.float32)
    dims = (((0,), (0,)), ((), ()))
    for q, x_ref in enumerate((x0_ref, x1_ref, x2_ref, x3_ref)):
        y = jax.lax.dot_general(
            x_ref[...], eye, dims, preferred_element_type=jnp.float32)
        yb = y.astype(jnp.bfloat16)
        lo = lax.bitcast_convert_type(yb[:, 0:HW], jnp.uint16)
        hi = lax.bitcast_convert_type(yb[:, HW:D], jnp.uint16)
        word = lo.astype(jnp.int32) | (hi.astype(jnp.int32) << 16)
        o_ref[:, q * HW:(q + 1) * HW] = word


def _pack_table(tbl_t):
    d, v = tbl_t.shape
    q_off = QROWS // _TBLK
    last_blk = (v + _TBLK - 1) // _TBLK - 1

    def mk_map(q):
        return lambda i: (0, jnp.minimum(i + q * q_off, last_blk))

    return pl.pallas_call(
        _tr_body,
        grid=(q_off,),
        in_specs=[pl.BlockSpec((d, _TBLK), mk_map(q)) for q in range(4)],
        out_specs=pl.BlockSpec((_TBLK, 2 * D), lambda i: (i, 0)),
        out_shape=jax.ShapeDtypeStruct((QROWS, 2 * D), jnp.int32),
    )(tbl_t, tbl_t, tbl_t, tbl_t)


def kernel(pos_triplets, neg_triplets, node_em, edge_em):
    B = pos_triplets.shape[0]
    sc = _build_sc_kernel(B)
    partials = sc(
        pos_triplets[:, 0], pos_triplets[:, 1], pos_triplets[:, 2],
        neg_triplets[:, 0], neg_triplets[:, 1], neg_triplets[:, 2],
        _pack_table(node_em.T), _pack_table(edge_em.T))
    loss2d = pl.pallas_call(
        functools.partial(_finish_body, 1.0 / B),
        out_shape=jax.ShapeDtypeStruct((1, 1), jnp.float32),
    )(partials)
    return loss2d[0, 0]


# speedup vs baseline: 1.0018x; 1.0018x over previous
import functools

import jax
import jax.numpy as jnp
from jax import lax
from jax.experimental import pallas as pl
from jax.experimental.pallas import tpu as pltpu
from jax.experimental.pallas import tpu_sc as plsc

NC = 2
NS = 16
NW = NC * NS
L = 16
D = 64
QROWS = 1 << 18
QMASK = QROWS - 1
HW = D // 2


def _rsqrt16(x):
    i = plsc.bitcast(x, jnp.int32)
    z = plsc.bitcast(jnp.int32(0x5F3759DF) - (i >> 1), jnp.float32)
    for _ in range(3):
        z = z * (1.5 - 0.5 * x * z * z)
    return z


def _build_sc_kernel(B):
    CH = B // NW
    NSUB = 2
    SUB = CH // NSUB
    NG = SUB // L

    mesh = plsc.VectorSubcoreMesh(
        core_axis_name="c", subcore_axis_name="s",
        num_cores=NC, num_subcores=NS)

    @functools.partial(
        pl.kernel,
        out_type=jax.ShapeDtypeStruct((NW, L), jnp.float32),
        mesh=mesh,
        scratch_types=[
            pltpu.VMEM((SUB,), jnp.int32),
            pltpu.VMEM((SUB,), jnp.int32),
            pltpu.VMEM((SUB,), jnp.int32),
            pltpu.VMEM((SUB,), jnp.int32),
            pltpu.VMEM((SUB,), jnp.int32),
            pltpu.VMEM((SUB,), jnp.int32),
            pltpu.VMEM((SUB,), jnp.int32),
            pltpu.VMEM((SUB, 2 * D), jnp.int32),
            pltpu.VMEM((SUB, 2 * D), jnp.int32),
            pltpu.VMEM((SUB, 2 * D), jnp.int32),
            pltpu.VMEM((CH,), jnp.float32),
            pltpu.VMEM((L,), jnp.float32),
            pltpu.SemaphoreType.DMA,
        ],
        compiler_params=pltpu.CompilerParams(
            needs_layout_passes=False, use_tc_tiling_on_sc=False),
    )
    def sc_kernel(ph, pr, pt, nh, nr, nt, node_pk, edge_pk, out,
                  raw_v, rh_v, rr_v, rt_v, ch_v, cr_v, ct_v,
                  h_v, r_v, t_v, pd_v, acc_v, sem):
        wid = lax.axis_index("s") * NC + lax.axis_index("c")

        def stage_idx(ix_hbm, base, row_v, cb_v):
            pltpu.sync_copy(ix_hbm.at[pl.ds(base, SUB)], raw_v)
            for k in range(SUB // L):
                v = raw_v[pl.ds(k * L, L)]
                row_v[pl.ds(k * L, L)] = v & QMASK
                cb_v[pl.ds(k * L, L)] = (v >> 18) << 5

        def gather_sub(hi, ri, ti, base):
            stage_idx(hi, base, rh_v, ch_v)
            stage_idx(ri, base, rr_v, cr_v)
            stage_idx(ti, base, rt_v, ct_v)
            c1 = pltpu.async_copy(node_pk.at[rh_v], h_v, sem)
            c2 = pltpu.async_copy(edge_pk.at[rr_v], r_v, sem)
            c3 = pltpu.async_copy(node_pk.at[rt_v], t_v, sem)
            c1.wait(); c2.wait(); c3.wait()

        def dist_group(g):
            row = g * L + lax.iota(jnp.int32, L)
            cbh = ch_v[pl.ds(g * L, L)]
            cbr = cr_v[pl.ds(g * L, L)]
            cbt = ct_v[pl.ds(g * L, L)]
            fmt = plsc.PackFormat.INTERLEAVED

            def jstep(w, ss):
                hw = plsc.load_gather(h_v, [row, cbh + w])
                rw = plsc.load_gather(r_v, [row, cbr + w])
                tw = plsc.load_gather(t_v, [row, cbt + w])
                ha, hb = plsc.unpack(plsc.bitcast(hw, jnp.bfloat16), format=fmt)
                ra, rb = plsc.unpack(plsc.bitcast(rw, jnp.bfloat16), format=fmt)
                ta, tb = plsc.unpack(plsc.bitcast(tw, jnp.bfloat16), format=fmt)
                d1 = ha + ra - ta
                d2 = hb + rb - tb
                return ss + d1 * d1 + d2 * d2

            ss = lax.fori_loop(0, HW, jstep, jnp.zeros((L,), jnp.float32))
            return ss * _rsqrt16(ss)

        base0 = wid * CH
        for s in range(NSUB):
            gather_sub(ph, pr, pt, base0 + s * SUB)

            def pos_body(g, carry, s=s):
                pd_v[pl.ds(s * SUB + g * L, L)] = dist_group(g)
                return carry

            lax.fori_loop(0, NG, pos_body, 0)

        acc = jnp.zeros((L,), jnp.float32)
        for s in range(NSUB):
            gather_sub(nh, nr, nt, base0 + s * SUB)

            def neg_body(g, a, s=s):
                nd = dist_group(g)
                pd = pd_v[pl.ds(s * SUB + g * L, L)]
                return a + jnp.maximum(pd - nd, 0.0)

            acc = lax.fori_loop(0, NG, neg_body, acc)

        acc_v[...] = acc
        pltpu.sync_copy(acc_v, out.at[wid])

    return sc_kernel


def _finish_body(inv_b, p_ref, o_ref):
    o_ref[...] = jnp.reshape(jnp.sum(p_ref[...]) * inv_b, (1, 1))


_TBLK = 8192


def _tr_body(x0_ref, x1_ref, x2_ref, x3_ref, o_ref):
    d = x0_ref.shape[0]
    eye = (lax.broadcasted_iota(jnp.int32, (d, d), 0)
           == lax.broadcasted_iota(jnp.int32, (d, d), 1)).astype(jnp.float32)
    dims = (((0,), (0,)), ((), ()))
    for q, x_ref in enumerate((x0_ref, x1_ref, x2_ref, x3_ref)):
        y = jax.lax.dot_general(
            x_ref[...], eye, dims, preferred_element_type=jnp.float32)
        ybits = lax.bitcast_convert_type(y, jnp.int32) + jnp.int32(0x8000)
        lo = (ybits[:, 0:HW] >> 16) & jnp.int32(0xFFFF)
        hi = ybits[:, HW:D] & jnp.int32(-65536)
        o_ref[:, q * HW:(q + 1) * HW] = lo | hi


def _pack_table(tbl_t):
    d, v = tbl_t.shape
    q_off = QROWS // _TBLK
    last_blk = (v + _TBLK - 1) // _TBLK - 1

    def mk_map(q):
        return lambda i: (0, jnp.minimum(i + q * q_off, last_blk))

    return pl.pallas_call(
        _tr_body,
        grid=(q_off,),
        in_specs=[pl.BlockSpec((d, _TBLK), mk_map(q)) for q in range(4)],
        out_specs=pl.BlockSpec((_TBLK, 2 * D), lambda i: (i, 0)),
        out_shape=jax.ShapeDtypeStruct((QROWS, 2 * D), jnp.int32),
    )(tbl_t, tbl_t, tbl_t, tbl_t)


def kernel(pos_triplets, neg_triplets, node_em, edge_em):
    B = pos_triplets.shape[0]
    sc = _build_sc_kernel(B)
    partials = sc(
        pos_triplets[:, 0], pos_triplets[:, 1], pos_triplets[:, 2],
        neg_triplets[:, 0], neg_triplets[:, 1], neg_triplets[:, 2],
        _pack_table(node_em.T), _pack_table(edge_em.T))
    loss2d = pl.pallas_call(
        functools.partial(_finish_body, 1.0 / B),
        out_shape=jax.ShapeDtypeStruct((1, 1), jnp.float32),
    )(partials)
    return loss2d[0, 0]


# speedup vs baseline: 1.3170x; 1.3147x over previous
import functools

import jax
import jax.numpy as jnp
from jax import lax
from jax.experimental import pallas as pl
from jax.experimental.pallas import tpu as pltpu
from jax.experimental.pallas import tpu_sc as plsc

NC = 2
NS = 16
NW = NC * NS
L = 16
D = 64
PAIR = 1 << 19
VMASK = PAIR - 1


def _rsqrt16(x):
    i = plsc.bitcast(x, jnp.int32)
    z = plsc.bitcast(jnp.int32(0x5F3759DF) - (i >> 1), jnp.float32)
    for _ in range(3):
        z = z * (1.5 - 0.5 * x * z * z)
    return z


def _build_sc_kernel(B):
    CH = B // NW
    NSUB = 4
    SUB = CH // NSUB
    NG = SUB // L

    mesh = plsc.VectorSubcoreMesh(
        core_axis_name="c", subcore_axis_name="s",
        num_cores=NC, num_subcores=NS)

    idx_set = [pltpu.VMEM((SUB,), jnp.int32)] * 6
    buf_set = [pltpu.VMEM((SUB, 2 * D), jnp.float32)] * 3

    @functools.partial(
        pl.kernel,
        out_type=jax.ShapeDtypeStruct((NW, L), jnp.float32),
        mesh=mesh,
        scratch_types=(
            [pltpu.VMEM((SUB,), jnp.int32)]
            + idx_set + idx_set + buf_set + buf_set
            + [
                pltpu.VMEM((CH,), jnp.float32),
                pltpu.VMEM((L,), jnp.float32),
                pltpu.SemaphoreType.DMA,
                pltpu.SemaphoreType.DMA,
            ]
        ),
        compiler_params=pltpu.CompilerParams(
            needs_layout_passes=False, use_tc_tiling_on_sc=False),
    )
    def sc_kernel(ph, pr, pt, nh, nr, nt, node_pk, edge_pk, out,
                  raw_v,
                  rh0, rr0, rt0, ch0, cr0, ct0,
                  rh1, rr1, rt1, ch1, cr1, ct1,
                  h0, r0, t0, h1, r1, t1,
                  pd_v, acc_v, sem0, sem1):
        wid = lax.axis_index("s") * NC + lax.axis_index("c")
        sets = (
            ((rh0, rr0, rt0), (ch0, cr0, ct0), (h0, r0, t0), sem0),
            ((rh1, rr1, rt1), (ch1, cr1, ct1), (h1, r1, t1), sem1),
        )

        def stage_idx(ix_hbm, base, row_v, cb_v):
            pltpu.sync_copy(ix_hbm.at[pl.ds(base, SUB)], raw_v)
            for k in range(SUB // L):
                v = raw_v[pl.ds(k * L, L)]
                row_v[pl.ds(k * L, L)] = v & VMASK
                cb_v[pl.ds(k * L, L)] = (v >> 19) << 6

        def fire(chunk, k):
            (hi, ri, ti), base = chunk[1], chunk[2]
            rows, cbs, bufs, sem = sets[k]
            stage_idx(hi, base, rows[0], cbs[0])
            stage_idx(ri, base, rows[1], cbs[1])
            stage_idx(ti, base, rows[2], cbs[2])
            return (
                pltpu.async_copy(node_pk.at[rows[0]], bufs[0], sem),
                pltpu.async_copy(edge_pk.at[rows[1]], bufs[1], sem),
                pltpu.async_copy(node_pk.at[rows[2]], bufs[2], sem),
            )

        def dist_group(g, k):
            _, cbs, bufs, _ = sets[k]
            row = g * L + lax.iota(jnp.int32, L)
            cbh = cbs[0][pl.ds(g * L, L)]
            cbr = cbs[1][pl.ds(g * L, L)]
            cbt = cbs[2][pl.ds(g * L, L)]

            def jstep(j, ss):
                h = plsc.load_gather(bufs[0], [row, cbh + j])
                r = plsc.load_gather(bufs[1], [row, cbr + j])
                t = plsc.load_gather(bufs[2], [row, cbt + j])
                d = h + r - t
                return ss + d * d

            ss = lax.fori_loop(0, D, jstep, jnp.zeros((L,), jnp.float32))
            return ss * _rsqrt16(ss)

        base0 = wid * CH
        chunks = (
            [("pos", (ph, pr, pt), base0 + s * SUB, s) for s in range(NSUB)]
            + [("neg", (nh, nr, nt), base0 + s * SUB, s) for s in range(NSUB)]
        )

        acc = jnp.zeros((L,), jnp.float32)
        pending = fire(chunks[0], 0)
        for ci, chunk in enumerate(chunks):
            k = ci % 2
            if ci + 1 < len(chunks):
                nxt = fire(chunks[ci + 1], (ci + 1) % 2)
            for c in pending:
                c.wait()
            s = chunk[3]
            if chunk[0] == "pos":
                def pos_body(g, carry, s=s, k=k):
                    pd_v[pl.ds(s * SUB + g * L, L)] = dist_group(g, k)
                    return carry

                lax.fori_loop(0, NG, pos_body, 0)
            else:
                def neg_body(g, a, s=s, k=k):
                    nd = dist_group(g, k)
                    pd = pd_v[pl.ds(s * SUB + g * L, L)]
                    return a + jnp.maximum(pd - nd, 0.0)

                acc = lax.fori_loop(0, NG, neg_body, acc)
            if ci + 1 < len(chunks):
                pending = nxt

        acc_v[...] = acc
        pltpu.sync_copy(acc_v, out.at[wid])

    return sc_kernel


def _finish_body(inv_b, p_ref, o_ref):
    o_ref[...] = jnp.reshape(jnp.sum(p_ref[...]) * inv_b, (1, 1))


_TBLK = 16384


def _tr_body(xlo_ref, xhi_ref, o_ref):
    d = xlo_ref.shape[0]
    eye = (lax.broadcasted_iota(jnp.int32, (d, d), 0)
           == lax.broadcasted_iota(jnp.int32, (d, d), 1)).astype(jnp.float32)
    dims = (((0,), (0,)), ((), ()))
    o_ref[:, 0:d] = jax.lax.dot_general(
        xlo_ref[...], eye, dims, preferred_element_type=jnp.float32)
    o_ref[:, d:2 * d] = jax.lax.dot_general(
        xhi_ref[...], eye, dims, preferred_element_type=jnp.float32)


def _pack_table(tbl_t):
    d, v = tbl_t.shape
    hi_off = PAIR // _TBLK
    last_blk = (v + _TBLK - 1) // _TBLK - 1
    return pl.pallas_call(
        _tr_body,
        grid=(hi_off,),
        in_specs=[
            pl.BlockSpec((d, _TBLK), lambda i: (0, i)),
            pl.BlockSpec((d, _TBLK),
                         lambda i: (0, jnp.minimum(i + hi_off, last_blk))),
        ],
        out_specs=pl.BlockSpec((_TBLK, 2 * d), lambda i: (i, 0)),
        out_shape=jax.ShapeDtypeStruct((PAIR, 2 * d), jnp.float32),
    )(tbl_t, tbl_t)


def kernel(pos_triplets, neg_triplets, node_em, edge_em):
    B = pos_triplets.shape[0]
    sc = _build_sc_kernel(B)
    partials = sc(
        pos_triplets[:, 0], pos_triplets[:, 1], pos_triplets[:, 2],
        neg_triplets[:, 0], neg_triplets[:, 1], neg_triplets[:, 2],
        _pack_table(node_em.T), _pack_table(edge_em.T))
    loss2d = pl.pallas_call(
        functools.partial(_finish_body, 1.0 / B),
        out_shape=jax.ShapeDtypeStruct((1, 1), jnp.float32),
    )(partials)
    return loss2d[0, 0]
